# Initial kernel scaffold; baseline (speedup 1.0000x reference)
import jax
import jax.numpy as jnp
from jax.experimental import pallas as pl


def kernel(z, ei, W, b):
    raise NotImplementedError("write your pallas kernel here")


# speedup vs baseline: 14.3432x; 14.3432x over previous
import jax
import jax.numpy as jnp
from jax import lax
from jax.experimental import pallas as pl
from jax.experimental.pallas import tpu as pltpu
from jax.experimental.pallas import tpu_sc as plsc

N = 50000
E = 800000
D = 64

NC = 2
NS = 16

CHUNK = 128
CPS = 14
SUP = 28
EDGES_PER_TILE = SUP * CPS * CHUNK
E_PAD = NS * EDGES_PER_TILE

HALF = 25088
N_PAD = NC * HALF
AGG_ROWS = HALF + 8
DUMP = HALF

HIST_W = 16

ROWS_PER_TILE = HALF // NS

WBC = 112
WBA = 392

_mesh = plsc.VectorSubcoreMesh(core_axis_name="c", subcore_axis_name="s")


def _hist_body(dst_hbm, zeros_hbm, hist_out, dst_v, ldst_v, ones_v, zb, hist_sh):
    cid = lax.axis_index("c")
    sid = lax.axis_index("s")

    lane = lax.iota(jnp.int32, 16)
    one_hot = jnp.where(lane == 0, 1.0, 0.0).astype(jnp.float32)

    def fill_ones(r, _):
        ones_v[r, :] = one_hot
        return 0
    lax.fori_loop(0, CHUNK, fill_ones, 0)

    pltpu.sync_copy(zeros_hbm, zb)
    hbase = sid * ROWS_PER_TILE

    def zrow(j, _):
        pltpu.sync_copy(zb, hist_sh.at[pl.ds(hbase + j * WBA, WBA)])
        return 0
    lax.fori_loop(0, 4, zrow, 0)
    plsc.subcore_barrier()

    row_base = cid * HALF
    ebase = sid * EDGES_PER_TILE

    def super_body(sup, _):
        sbase = ebase + sup * (CPS * CHUNK)
        pltpu.sync_copy(dst_hbm.at[pl.ds(sbase, CPS * CHUNK)], dst_v)

        def chunk_body(c, _):
            def lbody(i, _):
                dv = dst_v[pl.ds(c * CHUNK + i * 16, 16)]
                ld = dv - row_base
                ok = (ld >= 0) & (ld < HALF)
                ld = jnp.where(ok, ld, DUMP)
                ldst_v[pl.ds(i * 16, 16)] = ld
                return 0
            lax.fori_loop(0, 8, lbody, 0)
            pltpu.sync_copy(ones_v, hist_sh.at[ldst_v], add=True)
            return 0
        lax.fori_loop(0, CPS, chunk_body, 0)
        return 0
    lax.fori_loop(0, SUP, super_body, 0)
    plsc.subcore_barrier()

    def wrow(j, _):
        r = hbase + j * WBA
        pltpu.sync_copy(hist_sh.at[pl.ds(r, WBA)], zb)
        pltpu.sync_copy(zb, hist_out.at[pl.ds(row_base + r, WBA)])
        return 0
    lax.fori_loop(0, 4, wrow, 0)


_hist_call = pl.kernel(
    _hist_body,
    out_type=jax.ShapeDtypeStruct((N_PAD, HIST_W), jnp.float32),
    mesh=_mesh,
    compiler_params=pltpu.CompilerParams(use_tc_tiling_on_sc=False),
    scratch_types=[
        pltpu.VMEM((CPS * CHUNK,), jnp.int32),
        pltpu.VMEM((CHUNK,), jnp.int32),
        pltpu.VMEM((CHUNK, HIST_W), jnp.float32),
        pltpu.VMEM((WBA, HIST_W), jnp.float32),
        pltpu.VMEM_SHARED((AGG_ROWS, HIST_W), jnp.float32),
    ],
)


def _agg_body(y_hbm, src_hbm, dst_hbm, zeros_hbm, s_out,
              src_v, dst_v, ldst_v, ybuf, zb, agg_sh, sem0, sem1):
    cid = lax.axis_index("c")
    sid = lax.axis_index("s")
    sems = (sem0, sem1)

    pltpu.sync_copy(zeros_hbm, zb)
    rbase = sid * ROWS_PER_TILE

    def zrow(j, _):
        pltpu.sync_copy(zb, agg_sh.at[pl.ds(rbase + j * WBC, WBC)])
        return 0
    lax.fori_loop(0, 14, zrow, 0)
    plsc.subcore_barrier()

    row_base = cid * HALF
    ebase = sid * EDGES_PER_TILE

    def gather_start(c, b):
        idx = src_v.at[pl.ds(c * CHUNK, CHUNK)]
        pltpu.async_copy(y_hbm.at[idx], ybuf.at[b], sems[b])

    def gather_wait(c, b):
        idx = src_v.at[pl.ds(c * CHUNK, CHUNK)]
        pltpu.make_async_copy(y_hbm.at[idx], ybuf.at[b], sems[b]).wait()

    def super_body(sup, _):
        sbase = ebase + sup * (CPS * CHUNK)
        pltpu.sync_copy(src_hbm.at[pl.ds(sbase, CPS * CHUNK)], src_v)
        pltpu.sync_copy(dst_hbm.at[pl.ds(sbase, CPS * CHUNK)], dst_v)

        gather_start(0, 0)
        gather_start(1, 1)

        def pair_body(p, _):
            for b in range(2):
                c = p * 2 + b

                def lbody(i, _):
                    dv = dst_v[pl.ds(c * CHUNK + i * 16, 16)]
                    ld = dv - row_base
                    ok = (ld >= 0) & (ld < HALF)
                    ld = jnp.where(ok, ld, DUMP)
                    ldst_v[pl.ds(i * 16, 16)] = ld
                    return 0
                lax.fori_loop(0, 8, lbody, 0)
                gather_wait(c, b)
                pltpu.sync_copy(ybuf.at[b], agg_sh.at[ldst_v], add=True)
                nxt = c + 2

                @pl.when(nxt < CPS)
                def _():
                    gather_start(nxt, b)
            return 0
        lax.fori_loop(0, CPS // 2, pair_body, 0)
        return 0
    lax.fori_loop(0, SUP, super_body, 0)
    plsc.subcore_barrier()

    def wrow(j, _):
        r = rbase + j * WBC
        pltpu.sync_copy(agg_sh.at[pl.ds(r, WBC)], zb)
        pltpu.sync_copy(zb, s_out.at[pl.ds(cid * HALF + r, WBC)])
        return 0
    lax.fori_loop(0, 14, wrow, 0)


_agg_call = pl.kernel(
    _agg_body,
    out_type=jax.ShapeDtypeStruct((N_PAD, D), jnp.float32),
    mesh=_mesh,
    compiler_params=pltpu.CompilerParams(use_tc_tiling_on_sc=False),
    scratch_types=[
        pltpu.VMEM((CPS * CHUNK,), jnp.int32),
        pltpu.VMEM((CPS * CHUNK,), jnp.int32),
        pltpu.VMEM((CHUNK,), jnp.int32),
        pltpu.VMEM((2, CHUNK, D), jnp.float32),
        pltpu.VMEM((WBC, D), jnp.float32),
        pltpu.VMEM_SHARED((AGG_ROWS, D), jnp.float32),
        pltpu.SemaphoreType.DMA,
        pltpu.SemaphoreType.DMA,
    ],
)


BLK = 512


def _norm_body(hist_ref, z_ref, y_ref, dinv_ref):
    deg = hist_ref[:, 0:1] + 1.0
    dinv = lax.rsqrt(deg)
    y_ref[...] = dinv * z_ref[...]
    dinv_ref[...] = jnp.broadcast_to(dinv, (BLK, 8))


def _norm_call(hist, z):
    grid = (N_PAD // BLK,)
    return pl.pallas_call(
        _norm_body,
        grid=grid,
        in_specs=[
            pl.BlockSpec((BLK, HIST_W), lambda i: (i, 0)),
            pl.BlockSpec((BLK, D), lambda i: (i, 0)),
        ],
        out_specs=[
            pl.BlockSpec((BLK, D), lambda i: (i, 0)),
            pl.BlockSpec((BLK, 8), lambda i: (i, 0)),
        ],
        out_shape=[
            jax.ShapeDtypeStruct((N_PAD, D), jnp.float32),
            jax.ShapeDtypeStruct((N_PAD, 8), jnp.float32),
        ],
    )(hist, z)


def _out_body(s_ref, y_ref, dinv_ref, w_ref, b_ref, o_ref):
    dinv = dinv_ref[:, 0:1]
    agg = dinv * (s_ref[...] + y_ref[...])
    acc = jnp.dot(agg, w_ref[...], preferred_element_type=jnp.float32)
    o_ref[...] = jnp.maximum(acc + b_ref[...], 0.0)


def _out_call(s, y, dinv, W, b2):
    grid = (N_PAD // BLK,)
    return pl.pallas_call(
        _out_body,
        grid=grid,
        in_specs=[
            pl.BlockSpec((BLK, D), lambda i: (i, 0)),
            pl.BlockSpec((BLK, D), lambda i: (i, 0)),
            pl.BlockSpec((BLK, 8), lambda i: (i, 0)),
            pl.BlockSpec((D, D), lambda i: (0, 0)),
            pl.BlockSpec((1, D), lambda i: (0, 0)),
        ],
        out_specs=pl.BlockSpec((BLK, D), lambda i: (i, 0)),
        out_shape=jax.ShapeDtypeStruct((N, D), jnp.float32),
    )(s, y, dinv, W, b2)


@jax.jit
def kernel(z, ei, W, b):
    src = ei[0].astype(jnp.int32)
    dst = ei[1].astype(jnp.int32)
    pad = E_PAD - E
    src = jnp.concatenate([src, jnp.zeros((pad,), jnp.int32)])
    dst = jnp.concatenate([dst, jnp.full((pad,), -1, jnp.int32)])

    zeros_a = jnp.zeros((WBA, HIST_W), jnp.float32)
    zeros_c = jnp.zeros((WBC, D), jnp.float32)
    b2 = b.reshape(1, D)

    hist = _hist_call(dst, zeros_a)
    y, dinv = _norm_call(hist, z)
    s = _agg_call(y, src, dst, zeros_c)
    return _out_call(s, y, dinv, W, b2)


# speedup vs baseline: 20.5765x; 1.4346x over previous
import jax
import jax.numpy as jnp
from jax import lax
from jax.experimental import pallas as pl
from jax.experimental.pallas import tpu as pltpu
from jax.experimental.pallas import tpu_sc as plsc

N = 50000
E = 800000
D = 64

NC = 2
NS = 16

CHUNK = 128
CPS = 14
SUP = 28
EDGES_PER_TILE = SUP * CPS * CHUNK
E_PAD = NS * EDGES_PER_TILE

HALF = 25088
N_PAD = NC * HALF
AGG_ROWS = HALF + 8
DUMP = HALF

HIST_W = 8
HIST_ROWS = N_PAD + 8
DUMP_A = N_PAD
SUP_A = 14

ROWS_PER_TILE = HALF // NS
HB_PER_TILE = N_PAD // NS

WBC = 112
WBA = 392

_mesh = plsc.VectorSubcoreMesh(core_axis_name="c", subcore_axis_name="s")


def _hist_body(dst_hbm, zeros_hbm, ones_hbm, hist_out, dst_v, ldst_v, ones_v,
               zb, hist_sh, ssem):
    cid = lax.axis_index("c")
    sid = lax.axis_index("s")

    pltpu.sync_copy(ones_hbm, ones_v)

    pltpu.sync_copy(zeros_hbm, zb)
    hbase = sid * HB_PER_TILE

    def zrow(j, _):
        pltpu.sync_copy(zb, hist_sh.at[pl.ds(hbase + j * WBA, WBA)])
        return 0
    lax.fori_loop(0, 8, zrow, 0)
    plsc.subcore_barrier()

    ebase = cid * (E_PAD // 2) + sid * (SUP_A * CPS * CHUNK)

    def fire(sb, c):
        pltpu.async_copy(ones_v, hist_sh.at[ldst_v.at[sb, c]], ssem, add=True)

    def drain(sb, c):
        pltpu.make_async_copy(
            ones_v, hist_sh.at[ldst_v.at[sb, c]], ssem).wait()

    def super_body(sup, _):
        sb = sup % 2
        sbase = ebase + sup * (CPS * CHUNK)
        pltpu.sync_copy(dst_hbm.at[pl.ds(sbase, CPS * CHUNK)], dst_v)

        @pl.when(sup >= 2)
        def _():
            def dr(c, _):
                drain(sb, c)
                return 0
            lax.fori_loop(0, CPS, dr, 0)

        def lbody(i, _):
            dv = dst_v[pl.ds(i * 16, 16)]
            ld = jnp.where(dv >= 0, dv, DUMP_A)
            ldst_v[sb, i // 8, pl.ds((i % 8) * 16, 16)] = ld
            return 0
        lax.fori_loop(0, CPS * 8, lbody, 0)

        def fr(c, _):
            fire(sb, c)
            return 0
        lax.fori_loop(0, CPS, fr, 0)
        return 0
    lax.fori_loop(0, SUP_A, super_body, 0)

    def dr_tail(i, _):
        drain((SUP_A - 2 + i // CPS) % 2, i % CPS)
        return 0
    lax.fori_loop(0, 2 * CPS, dr_tail, 0)
    plsc.subcore_barrier()

    def wrow(j, _):
        r = hbase + j * WBA
        pltpu.sync_copy(hist_sh.at[pl.ds(r, WBA)], zb)
        pltpu.sync_copy(zb, hist_out.at[cid, pl.ds(r, WBA)])
        return 0
    lax.fori_loop(0, 8, wrow, 0)


_hist_call = pl.kernel(
    _hist_body,
    out_type=jax.ShapeDtypeStruct((NC, N_PAD, HIST_W), jnp.float32),
    mesh=_mesh,
    compiler_params=pltpu.CompilerParams(use_tc_tiling_on_sc=False),
    scratch_types=[
        pltpu.VMEM((CPS * CHUNK,), jnp.int32),
        pltpu.VMEM((2, CPS, CHUNK), jnp.int32),
        pltpu.VMEM((CHUNK, HIST_W), jnp.float32),
        pltpu.VMEM((WBA, HIST_W), jnp.float32),
        pltpu.VMEM_SHARED((HIST_ROWS, HIST_W), jnp.float32),
        pltpu.SemaphoreType.DMA,
    ],
)


def _agg_body(y_hbm, src_hbm, dst_hbm, zeros_hbm, s_out,
              src_v, dst_v, ldst_v, ybuf, zb, agg_sh, sem0, sem1):
    cid = lax.axis_index("c")
    sid = lax.axis_index("s")
    sems = (sem0, sem1)

    pltpu.sync_copy(zeros_hbm, zb)
    rbase = sid * ROWS_PER_TILE

    def zrow(j, _):
        pltpu.sync_copy(zb, agg_sh.at[pl.ds(rbase + j * WBC, WBC)])
        return 0
    lax.fori_loop(0, 14, zrow, 0)
    plsc.subcore_barrier()

    row_base = cid * HALF
    ebase = sid * EDGES_PER_TILE

    def gather_start(c, b):
        idx = src_v.at[pl.ds(c * CHUNK, CHUNK)]
        pltpu.async_copy(y_hbm.at[idx], ybuf.at[b], sems[b])

    def gather_wait(c, b):
        idx = src_v.at[pl.ds(c * CHUNK, CHUNK)]
        pltpu.make_async_copy(y_hbm.at[idx], ybuf.at[b], sems[b]).wait()

    def super_body(sup, _):
        sbase = ebase + sup * (CPS * CHUNK)
        pltpu.sync_copy(src_hbm.at[pl.ds(sbase, CPS * CHUNK)], src_v)
        pltpu.sync_copy(dst_hbm.at[pl.ds(sbase, CPS * CHUNK)], dst_v)

        gather_start(0, 0)
        gather_start(1, 1)

        def pair_body(p, _):
            for b in range(2):
                c = p * 2 + b

                def lbody(i, _):
                    dv = dst_v[pl.ds(c * CHUNK + i * 16, 16)]
                    ld = dv - row_base
                    ok = (ld >= 0) & (ld < HALF)
                    ld = jnp.where(ok, ld, DUMP)
                    ldst_v[pl.ds(i * 16, 16)] = ld
                    return 0
                lax.fori_loop(0, 8, lbody, 0)
                gather_wait(c, b)
                pltpu.sync_copy(ybuf.at[b], agg_sh.at[ldst_v], add=True)
                nxt = c + 2

                @pl.when(nxt < CPS)
                def _():
                    gather_start(nxt, b)
            return 0
        lax.fori_loop(0, CPS // 2, pair_body, 0)
        return 0
    lax.fori_loop(0, SUP, super_body, 0)
    plsc.subcore_barrier()

    def wrow(j, _):
        r = rbase + j * WBC
        pltpu.sync_copy(agg_sh.at[pl.ds(r, WBC)], zb)
        pltpu.sync_copy(zb, s_out.at[pl.ds(cid * HALF + r, WBC)])
        return 0
    lax.fori_loop(0, 14, wrow, 0)


_agg_call = pl.kernel(
    _agg_body,
    out_type=jax.ShapeDtypeStruct((N_PAD, D), jnp.float32),
    mesh=_mesh,
    compiler_params=pltpu.CompilerParams(use_tc_tiling_on_sc=False),
    scratch_types=[
        pltpu.VMEM((CPS * CHUNK,), jnp.int32),
        pltpu.VMEM((CPS * CHUNK,), jnp.int32),
        pltpu.VMEM((CHUNK,), jnp.int32),
        pltpu.VMEM((2, CHUNK, D), jnp.float32),
        pltpu.VMEM((WBC, D), jnp.float32),
        pltpu.VMEM_SHARED((AGG_ROWS, D), jnp.float32),
        pltpu.SemaphoreType.DMA,
        pltpu.SemaphoreType.DMA,
    ],
)


BLK = 512


def _norm_body(hist_ref, z_ref, y_ref, dinv_ref):
    deg = hist_ref[0, :, 0:1] + hist_ref[1, :, 0:1] + 1.0
    dinv = lax.rsqrt(deg)
    y_ref[...] = dinv * z_ref[...]
    dinv_ref[...] = jnp.broadcast_to(dinv, (BLK, 8))


def _norm_call(hist, z):
    grid = (N_PAD // BLK,)
    return pl.pallas_call(
        _norm_body,
        grid=grid,
        in_specs=[
            pl.BlockSpec((NC, BLK, HIST_W), lambda i: (0, i, 0)),
            pl.BlockSpec((BLK, D), lambda i: (i, 0)),
        ],
        out_specs=[
            pl.BlockSpec((BLK, D), lambda i: (i, 0)),
            pl.BlockSpec((BLK, 8), lambda i: (i, 0)),
        ],
        out_shape=[
            jax.ShapeDtypeStruct((N_PAD, D), jnp.float32),
            jax.ShapeDtypeStruct((N_PAD, 8), jnp.float32),
        ],
    )(hist, z)


def _out_body(s_ref, y_ref, dinv_ref, w_ref, b_ref, o_ref):
    dinv = dinv_ref[:, 0:1]
    agg = dinv * (s_ref[...] + y_ref[...])
    acc = jnp.dot(agg, w_ref[...], preferred_element_type=jnp.float32)
    o_ref[...] = jnp.maximum(acc + b_ref[...], 0.0)


def _out_call(s, y, dinv, W, b2):
    grid = (N_PAD // BLK,)
    return pl.pallas_call(
        _out_body,
        grid=grid,
        in_specs=[
            pl.BlockSpec((BLK, D), lambda i: (i, 0)),
            pl.BlockSpec((BLK, D), lambda i: (i, 0)),
            pl.BlockSpec((BLK, 8), lambda i: (i, 0)),
            pl.BlockSpec((D, D), lambda i: (0, 0)),
            pl.BlockSpec((1, D), lambda i: (0, 0)),
        ],
        out_specs=pl.BlockSpec((BLK, D), lambda i: (i, 0)),
        out_shape=jax.ShapeDtypeStruct((N, D), jnp.float32),
    )(s, y, dinv, W, b2)


@jax.jit
def kernel(z, ei, W, b):
    src = ei[0].astype(jnp.int32)
    dst = ei[1].astype(jnp.int32)
    pad = E_PAD - E
    src = jnp.concatenate([src, jnp.zeros((pad,), jnp.int32)])
    dst = jnp.concatenate([dst, jnp.full((pad,), -1, jnp.int32)])

    zeros_a = jnp.zeros((WBA, HIST_W), jnp.float32)
    zeros_c = jnp.zeros((WBC, D), jnp.float32)
    b2 = b.reshape(1, D)

    ones_a = jnp.zeros((CHUNK, HIST_W), jnp.float32).at[:, 0].set(1.0)
    hist = _hist_call(dst, zeros_a, ones_a)
    y, dinv = _norm_call(hist, z)
    s = _agg_call(y, src, dst, zeros_c)
    return _out_call(s, y, dinv, W, b2)
